# Initial kernel scaffold; baseline (speedup 1.0000x reference)
import jax
import jax.numpy as jnp
from jax.experimental import pallas as pl


def kernel(qk, v, rotations):
    raise NotImplementedError("write your pallas kernel here")


# speedup vs baseline: 5.4527x; 5.4527x over previous
import functools

import jax
import jax.numpy as jnp
from jax import lax
from jax.experimental import pallas as pl
from jax.experimental.pallas import tpu as pltpu
from jax.experimental.pallas import tpu_sc as plsc

B, T, D = 16, 2048, 64
H = 8
NBUCK = 32
BS = 64
C = H * NBUCK
NT = H * T
NW = 32
TPW = (B * H) // NW
SELF_VAL = -50000.0


def _hash_body(qk_ref, rot_ref, out_ref):
    x = qk_ref[0]
    r = jnp.dot(x, rot_ref[...], preferred_element_type=jnp.float32)
    iota32 = lax.broadcasted_iota(jnp.int32, (T, 2 * NBUCK // 2), 1)
    rows = []
    for h in range(H):
        sub = r[:, h * 16:(h + 1) * 16]
        full = jnp.concatenate([sub, -sub], axis=1)
        m = jnp.max(full, axis=1, keepdims=True)
        am = jnp.min(jnp.where(full == m, iota32, NBUCK), axis=1)
        rows.append(am.astype(jnp.int32).reshape(1, T))
    out_ref[0] = jnp.concatenate(rows, axis=0)


def _hash_call(qk, rot2):
    return pl.pallas_call(
        _hash_body,
        grid=(B,),
        in_specs=[
            pl.BlockSpec((1, T, D), lambda b: (b, 0, 0)),
            pl.BlockSpec((D, H * 16), lambda b: (0, 0)),
        ],
        out_specs=pl.BlockSpec((1, H, T), lambda b: (b, 0, 0)),
        out_shape=jax.ShapeDtypeStruct((B, H, T), jnp.int32),
    )(qk, rot2)


def _sort_gather_kernel(buck_hbm, qk_hbm, v_hbm,
                        st_hbm, pos_hbm, sqk_hbm, sv_hbm,
                        buk_v, rank_v, st_v, idxg_v, pos_v,
                        cnt_v, bs_v, rows_v, sem):
    wid = lax.axis_index("c") * 16 + lax.axis_index("s")
    idx16 = lax.iota(jnp.int32, 16)
    zeros16 = jnp.zeros((16,), jnp.int32)

    def task_body(i, _):
        tid = wid * TPW + i
        b = tid // H
        h = tid % H
        pltpu.sync_copy(buck_hbm.at[b, h], buk_v)
        cnt_v[pl.ds(0, 16)] = zeros16
        cnt_v[pl.ds(16, 16)] = zeros16

        def blk1(blk, _c):
            bvec = buk_v[pl.ds(blk * 16, 16)]
            key = bvec * 16 + idx16
            ks, vs = plsc.sort_key_val(key, idx16)
            bs = lax.shift_right_logical(ks, 4)
            bs_v[...] = bs
            prev = plsc.load_gather(bs_v, [jnp.maximum(idx16 - 1, 0)])
            is_start = (idx16 == 0) | (bs != prev)
            start_idx = plsc.cummax(jnp.where(is_start, idx16, 0))
            rnk = (idx16 - start_idx) + plsc.load_gather(cnt_v, [bs])
            nxt = plsc.load_gather(bs_v, [jnp.minimum(idx16 + 1, 15)])
            is_end = (idx16 == 15) | (bs != nxt)
            plsc.store_scatter(cnt_v, [bs], rnk + 1, mask=is_end)
            plsc.store_scatter(rank_v, [blk * 16 + vs], rnk)
            return _c

        lax.fori_loop(0, T // 16, blk1, 0)

        c0 = cnt_v[pl.ds(0, 16)]
        c1 = cnt_v[pl.ds(16, 16)]
        t0 = jnp.sum(c0)
        base0 = plsc.cumsum(c0) - c0
        base1 = plsc.cumsum(c1) + t0 - c1
        cnt_v[pl.ds(0, 16)] = base0
        cnt_v[pl.ds(16, 16)] = base1

        def blk2(blk, _c):
            bvec = buk_v[pl.ds(blk * 16, 16)]
            rnk = rank_v[pl.ds(blk * 16, 16)]
            ploc = plsc.load_gather(cnt_v, [bvec]) + rnk
            tvec = blk * 16 + idx16
            pos_v[pl.ds(blk * 16, 16)] = ploc + h * T
            plsc.store_scatter(st_v, [ploc], tvec)
            plsc.store_scatter(idxg_v, [ploc], tvec + b * T)
            return _c

        lax.fori_loop(0, T // 16, blk2, 0)

        pltpu.sync_copy(pos_v, pos_hbm.at[b, h])
        pltpu.sync_copy(st_v, st_hbm.at[b, pl.ds(h * T, T)])

        for cch in range(4):
            idx_sl = idxg_v.at[pl.ds(cch * 512, 512)]
            pltpu.async_copy(qk_hbm.at[idx_sl], rows_v, sem).wait()
            pltpu.sync_copy(rows_v, sqk_hbm.at[b, pl.ds(h * T + cch * 512, 512)])
            pltpu.async_copy(v_hbm.at[idx_sl], rows_v, sem).wait()
            pltpu.sync_copy(rows_v, sv_hbm.at[b, pl.ds(h * T + cch * 512, 512)])
        return _

    lax.fori_loop(0, TPW, task_body, 0)


def _sort_gather_call(buck, qk_flat, v_flat):
    mesh = plsc.VectorSubcoreMesh(core_axis_name="c", subcore_axis_name="s")
    fn = functools.partial(
        pl.kernel,
        mesh=mesh,
        compiler_params=pltpu.CompilerParams(
            needs_layout_passes=False, use_tc_tiling_on_sc=False),
        out_type=[
            jax.ShapeDtypeStruct((B, NT), jnp.int32),
            jax.ShapeDtypeStruct((B, H, T), jnp.int32),
            jax.ShapeDtypeStruct((B, NT, D), jnp.float32),
            jax.ShapeDtypeStruct((B, NT, D), jnp.float32),
        ],
        scratch_types=[
            pltpu.VMEM((T,), jnp.int32),
            pltpu.VMEM((T,), jnp.int32),
            pltpu.VMEM((T,), jnp.int32),
            pltpu.VMEM((T,), jnp.int32),
            pltpu.VMEM((T,), jnp.int32),
            pltpu.VMEM((32,), jnp.int32),
            pltpu.VMEM((16,), jnp.int32),
            pltpu.VMEM((512, D), jnp.float32),
            pltpu.SemaphoreType.DMA,
        ],
    )(_sort_gather_kernel)
    return fn(buck, qk_flat, v_flat)


def _attn_body(qc, qp, vc, vp, tc_, tp_, so_ref, sl_ref):
    mv = -jnp.finfo(jnp.float32).max
    for j in range(8):
        q = qc[0, j]
        kprev = qp[0, 7] if j == 0 else qc[0, j - 1]
        vprev = vp[0, 7] if j == 0 else vc[0, j - 1]
        tprev = tp_[0, 7] if j == 0 else tc_[0, j - 1]
        k2 = jnp.concatenate([q, kprev], axis=0)
        v2 = jnp.concatenate([vc[0, j], vprev], axis=0)
        kt = jnp.concatenate([tc_[0, j], tprev], axis=0)
        nrm = jnp.sqrt(jnp.sum(k2 * k2, axis=1, keepdims=True))
        k2n = k2 / jnp.maximum(nrm, 1e-12)
        dots = lax.dot_general(q, k2n, (((1,), (1,)), ((), ())),
                               preferred_element_type=jnp.float32) * 0.125
        qt = tc_[0, j].reshape(64, 1)
        ktr = kt.reshape(1, 128)
        dots = jnp.where(qt < ktr, mv, dots)
        dots = jnp.where(qt == ktr, SELF_VAL, dots)
        mx = jnp.max(dots, axis=1, keepdims=True)
        ex = jnp.exp(dots - mx)
        s = jnp.sum(ex, axis=1, keepdims=True)
        o = lax.dot_general(ex, v2, (((1,), (0,)), ((), ())),
                            preferred_element_type=jnp.float32) / s
        so_ref[0, j] = o
        sl_ref[0, j] = (jnp.log(s) + mx).reshape(64)


def _attn_call(sqk4, sv4, bqt):
    NI = C // 8
    cur4 = pl.BlockSpec((1, 8, BS, D), lambda b, i: (b, i, 0, 0))
    prev4 = pl.BlockSpec((1, 8, BS, D), lambda b, i: (b, (i + NI - 1) % NI, 0, 0))
    curt = pl.BlockSpec((1, 8, BS), lambda b, i: (b, i, 0))
    prevt = pl.BlockSpec((1, 8, BS), lambda b, i: (b, (i + NI - 1) % NI, 0))
    return pl.pallas_call(
        _attn_body,
        grid=(B, NI),
        in_specs=[cur4, prev4, cur4, prev4, curt, prevt],
        out_specs=[
            pl.BlockSpec((1, 8, BS, D), lambda b, i: (b, i, 0, 0)),
            pl.BlockSpec((1, 8, BS), lambda b, i: (b, i, 0)),
        ],
        out_shape=[
            jax.ShapeDtypeStruct((B, C, BS, D), jnp.float32),
            jax.ShapeDtypeStruct((B, C, BS), jnp.float32),
        ],
    )(sqk4, sqk4, sv4, sv4, bqt, bqt)


def _unsort_kernel(pos_hbm, so_hbm, slog_hbm,
                   ou_hbm, lu_hbm,
                   pos_v, slog_v, lbuf_v, idxg_v, rows_v, sem):
    wid = lax.axis_index("c") * 16 + lax.axis_index("s")

    def task_body(i, _):
        tid = wid * TPW + i
        b = tid // H
        h = tid % H
        pltpu.sync_copy(pos_hbm.at[b, h], pos_v)
        pltpu.sync_copy(slog_hbm.at[b], slog_v)

        def blk(blk_i, _c):
            p = pos_v[pl.ds(blk_i * 16, 16)]
            lbuf_v[pl.ds(blk_i * 16, 16)] = plsc.load_gather(slog_v, [p])
            idxg_v[pl.ds(blk_i * 16, 16)] = p + b * NT
            return _c

        lax.fori_loop(0, T // 16, blk, 0)
        pltpu.sync_copy(lbuf_v, lu_hbm.at[b, h])
        for cch in range(4):
            idx_sl = idxg_v.at[pl.ds(cch * 512, 512)]
            pltpu.async_copy(so_hbm.at[idx_sl], rows_v, sem).wait()
            pltpu.sync_copy(rows_v, ou_hbm.at[b, h, pl.ds(cch * 512, 512)])
        return _

    lax.fori_loop(0, TPW, task_body, 0)


def _unsort_call(pos, so_flat, slog):
    mesh = plsc.VectorSubcoreMesh(core_axis_name="c", subcore_axis_name="s")
    fn = functools.partial(
        pl.kernel,
        mesh=mesh,
        compiler_params=pltpu.CompilerParams(
            needs_layout_passes=False, use_tc_tiling_on_sc=False),
        out_type=[
            jax.ShapeDtypeStruct((B, H, T, D), jnp.float32),
            jax.ShapeDtypeStruct((B, H, T), jnp.float32),
        ],
        scratch_types=[
            pltpu.VMEM((T,), jnp.int32),
            pltpu.VMEM((NT,), jnp.float32),
            pltpu.VMEM((T,), jnp.float32),
            pltpu.VMEM((T,), jnp.int32),
            pltpu.VMEM((512, D), jnp.float32),
            pltpu.SemaphoreType.DMA,
        ],
    )(_unsort_kernel)
    return fn(pos, so_flat, slog)


def _combine_body(l_ref, o_ref, out_ref):
    l = l_ref[0]
    mx = jnp.max(l, axis=0, keepdims=True)
    w = jnp.exp(l - mx)
    w = w / jnp.sum(w, axis=0, keepdims=True)
    acc = o_ref[0, 0] * w[0].reshape(T, 1)
    for h in range(1, H):
        acc = acc + o_ref[0, h] * w[h].reshape(T, 1)
    out_ref[0] = acc


def _combine_call(l_uns, o_uns):
    return pl.pallas_call(
        _combine_body,
        grid=(B,),
        in_specs=[
            pl.BlockSpec((1, H, T), lambda b: (b, 0, 0)),
            pl.BlockSpec((1, H, T, D), lambda b: (b, 0, 0, 0)),
        ],
        out_specs=pl.BlockSpec((1, T, D), lambda b: (b, 0, 0)),
        out_shape=jax.ShapeDtypeStruct((B, T, D), jnp.float32),
    )(l_uns, o_uns)


def kernel(qk, v, rotations):
    rot2 = rotations[0].reshape(D, H * 16)
    buck = _hash_call(qk, rot2)
    st, pos, sqk, sv = _sort_gather_call(
        buck, qk.reshape(B * T, D), v.reshape(B * T, D))
    bqt = st.astype(jnp.float32).reshape(B, C, BS)
    so, slog = _attn_call(sqk.reshape(B, C, BS, D), sv.reshape(B, C, BS, D), bqt)
    o_uns, l_uns = _unsort_call(pos, so.reshape(B * NT, D), slog.reshape(B, NT))
    return _combine_call(l_uns, o_uns)


# speedup vs baseline: 7.6685x; 1.4064x over previous
import functools

import jax
import jax.numpy as jnp
from jax import lax
from jax.experimental import pallas as pl
from jax.experimental.pallas import tpu as pltpu
from jax.experimental.pallas import tpu_sc as plsc

B, T, D = 16, 2048, 64
H = 8
NBUCK = 32
BS = 64
C = H * NBUCK
NT = H * T
NW = 32
TPW = (B * H) // NW
SELF_VAL = -50000.0


def _hash_body(qk_ref, rot_ref, out_ref):
    x = qk_ref[0]
    rT = lax.dot_general(rot_ref[...], x, (((0,), (1,)), ((), ())),
                         preferred_element_type=jnp.float32)
    iota32 = lax.broadcasted_iota(jnp.int32, (NBUCK, T), 0)
    rows = []
    for h in range(H):
        sub = rT[h * 16:(h + 1) * 16]
        seg = jnp.concatenate([sub, -sub], axis=0)
        m = jnp.max(seg, axis=0, keepdims=True)
        am = jnp.min(jnp.where(seg == m, iota32, NBUCK), axis=0, keepdims=True)
        rows.append(am)
    out_ref[0] = jnp.concatenate(rows, axis=0)


def _hash_call(qk, rot2):
    return pl.pallas_call(
        _hash_body,
        grid=(B,),
        in_specs=[
            pl.BlockSpec((1, T, D), lambda b: (b, 0, 0)),
            pl.BlockSpec((D, H * 16), lambda b: (0, 0)),
        ],
        out_specs=pl.BlockSpec((1, H, T), lambda b: (b, 0, 0)),
        out_shape=jax.ShapeDtypeStruct((B, H, T), jnp.int32),
    )(qk, rot2)


def _sort_gather_kernel(buck_hbm, qk_hbm, v_hbm,
                        st_hbm, pos_hbm, sqk_hbm, sv_hbm,
                        buk_v, rank_v, st_v, idxg_v, pos_v,
                        cnt_v, bs_v, rows_v, sem):
    wid = lax.axis_index("c") * 16 + lax.axis_index("s")
    idx16 = lax.iota(jnp.int32, 16)
    zeros16 = jnp.zeros((16,), jnp.int32)

    def task_body(i, _):
        tid = wid * TPW + i
        b = tid // H
        h = tid % H
        pltpu.sync_copy(buck_hbm.at[b, h], buk_v)
        cnt_v[pl.ds(0, 16)] = zeros16
        cnt_v[pl.ds(16, 16)] = zeros16

        def blk1(blk, _c):
            bvec = buk_v[pl.ds(blk * 16, 16)]
            key = bvec * 16 + idx16
            ks, vs = plsc.sort_key_val(key, idx16)
            bs = lax.shift_right_logical(ks, 4)
            bs_v[...] = bs
            prev = plsc.load_gather(bs_v, [jnp.maximum(idx16 - 1, 0)])
            is_start = (idx16 == 0) | (bs != prev)
            start_idx = plsc.cummax(jnp.where(is_start, idx16, 0))
            rnk = (idx16 - start_idx) + plsc.load_gather(cnt_v, [bs])
            nxt = plsc.load_gather(bs_v, [jnp.minimum(idx16 + 1, 15)])
            is_end = (idx16 == 15) | (bs != nxt)
            plsc.store_scatter(cnt_v, [bs], rnk + 1, mask=is_end)
            plsc.store_scatter(rank_v, [blk * 16 + vs], rnk)
            return _c

        lax.fori_loop(0, T // 16, blk1, 0)

        c0 = cnt_v[pl.ds(0, 16)]
        c1 = cnt_v[pl.ds(16, 16)]
        t0 = jnp.sum(c0)
        base0 = plsc.cumsum(c0) - c0
        base1 = plsc.cumsum(c1) + t0 - c1
        cnt_v[pl.ds(0, 16)] = base0
        cnt_v[pl.ds(16, 16)] = base1

        def blk2(blk, _c):
            bvec = buk_v[pl.ds(blk * 16, 16)]
            rnk = rank_v[pl.ds(blk * 16, 16)]
            ploc = plsc.load_gather(cnt_v, [bvec]) + rnk
            tvec = blk * 16 + idx16
            pos_v[pl.ds(blk * 16, 16)] = ploc + h * T
            plsc.store_scatter(st_v, [ploc], tvec)
            plsc.store_scatter(idxg_v, [ploc], tvec + b * T)
            return _c

        lax.fori_loop(0, T // 16, blk2, 0)

        pltpu.sync_copy(pos_v, pos_hbm.at[b, h])
        pltpu.sync_copy(st_v, st_hbm.at[b, pl.ds(h * T, T)])

        for cch in range(4):
            idx_sl = idxg_v.at[pl.ds(cch * 512, 512)]
            pltpu.async_copy(qk_hbm.at[idx_sl], rows_v, sem).wait()
            pltpu.sync_copy(rows_v, sqk_hbm.at[b, pl.ds(h * T + cch * 512, 512)])
            pltpu.async_copy(v_hbm.at[idx_sl], rows_v, sem).wait()
            pltpu.sync_copy(rows_v, sv_hbm.at[b, pl.ds(h * T + cch * 512, 512)])
        return _

    lax.fori_loop(0, TPW, task_body, 0)


def _sort_gather_call(buck, qk_flat, v_flat):
    mesh = plsc.VectorSubcoreMesh(core_axis_name="c", subcore_axis_name="s")
    fn = functools.partial(
        pl.kernel,
        mesh=mesh,
        compiler_params=pltpu.CompilerParams(
            needs_layout_passes=False, use_tc_tiling_on_sc=False),
        out_type=[
            jax.ShapeDtypeStruct((B, NT), jnp.int32),
            jax.ShapeDtypeStruct((B, H, T), jnp.int32),
            jax.ShapeDtypeStruct((B, NT, D), jnp.float32),
            jax.ShapeDtypeStruct((B, NT, D), jnp.float32),
        ],
        scratch_types=[
            pltpu.VMEM((T,), jnp.int32),
            pltpu.VMEM((T,), jnp.int32),
            pltpu.VMEM((T,), jnp.int32),
            pltpu.VMEM((T,), jnp.int32),
            pltpu.VMEM((T,), jnp.int32),
            pltpu.VMEM((32,), jnp.int32),
            pltpu.VMEM((16,), jnp.int32),
            pltpu.VMEM((512, D), jnp.float32),
            pltpu.SemaphoreType.DMA,
        ],
    )(_sort_gather_kernel)
    return fn(buck, qk_flat, v_flat)


def _attn_body(qc, qp1, vc, vp1, tq, tkc, tkp, so_ref, sl_ref):
    mv = -jnp.finfo(jnp.float32).max
    NCH = 8
    R = NCH * BS
    dots_list = []
    for j in range(NCH):
        q = qc[0, j]
        kprev = qp1[0, 0] if j == 0 else qc[0, j - 1]
        kmat = jnp.concatenate([q, kprev], axis=0)
        dots_list.append(
            lax.dot_general(q, kmat, (((1,), (1,)), ((), ())),
                            preferred_element_type=jnp.float32))
    dots = jnp.concatenate(dots_list, axis=0)

    ns = jnp.sum(qc[0] * qc[0], axis=2)
    nprev = jnp.sum(qp1[0, 0] * qp1[0, 0], axis=1).reshape(1, BS)
    nshift = jnp.concatenate([nprev, ns[:-1]], axis=0)
    nmat = jnp.concatenate([ns, nshift], axis=1)
    scale = 0.125 / jnp.maximum(jnp.sqrt(nmat), 1e-12)
    scale_b = jnp.broadcast_to(scale[:, None, :], (NCH, BS, 2 * BS)).reshape(R, 2 * BS)
    dots = dots * scale_b

    tcur = tkc[0, :, 0]
    tshift = jnp.concatenate([tkp[0, 0], tcur[:-1]], axis=0)
    ktm = jnp.concatenate([tcur, tshift], axis=1)
    ktb = jnp.broadcast_to(ktm[:, None, :], (NCH, BS, 2 * BS)).reshape(R, 2 * BS)
    qt = tq[0]
    dots = jnp.where(qt < ktb, mv, dots)
    dots = jnp.where(qt == ktb, SELF_VAL, dots)

    mx = jnp.max(dots, axis=1, keepdims=True)
    ex = jnp.exp(dots - mx)
    s = jnp.sum(ex, axis=1, keepdims=True)
    sl_ref[0] = jnp.log(s) + mx

    o_list = []
    for j in range(NCH):
        vprev = vp1[0, 0] if j == 0 else vc[0, j - 1]
        vmat = jnp.concatenate([vc[0, j], vprev], axis=0)
        o_list.append(
            lax.dot_general(ex[j * BS:(j + 1) * BS], vmat,
                            (((1,), (0,)), ((), ())),
                            preferred_element_type=jnp.float32))
    so_ref[0] = jnp.concatenate(o_list, axis=0) / s


def _attn_call(sqk4, sv4, tq, tk):
    NI = C // 8
    R = 8 * BS
    cur4 = pl.BlockSpec((1, 8, BS, D), lambda b, i: (b, i, 0, 0))
    prev1 = pl.BlockSpec((1, 1, BS, D), lambda b, i: (b, (i * 8 + C - 1) % C, 0, 0))
    tqs = pl.BlockSpec((1, R, 1), lambda b, i: (b, i, 0))
    tkc = pl.BlockSpec((1, 8, 1, BS), lambda b, i: (b, i, 0, 0))
    tkp = pl.BlockSpec((1, 1, 1, BS), lambda b, i: (b, (i * 8 + C - 1) % C, 0, 0))
    return pl.pallas_call(
        _attn_body,
        grid=(B, NI),
        in_specs=[cur4, prev1, cur4, prev1, tqs, tkc, tkp],
        out_specs=[
            pl.BlockSpec((1, R, D), lambda b, i: (b, i, 0)),
            pl.BlockSpec((1, R, 1), lambda b, i: (b, i, 0)),
        ],
        out_shape=[
            jax.ShapeDtypeStruct((B, NT, D), jnp.float32),
            jax.ShapeDtypeStruct((B, NT, 1), jnp.float32),
        ],
    )(sqk4, sqk4, sv4, sv4, tq, tk, tk)


def _unsort_kernel(pos_hbm, so_hbm, slog_hbm,
                   ou_hbm, lu_hbm,
                   pos_v, slog_v, lbuf_v, idxg_v, rows_v, sem):
    wid = lax.axis_index("c") * 16 + lax.axis_index("s")

    def task_body(i, _):
        tid = wid * TPW + i
        b = tid // H
        h = tid % H
        pltpu.sync_copy(pos_hbm.at[b, h], pos_v)
        pltpu.sync_copy(slog_hbm.at[b], slog_v)

        def blk(blk_i, _c):
            p = pos_v[pl.ds(blk_i * 16, 16)]
            lbuf_v[pl.ds(blk_i * 16, 16)] = plsc.load_gather(slog_v, [p])
            idxg_v[pl.ds(blk_i * 16, 16)] = p + b * NT
            return _c

        lax.fori_loop(0, T // 16, blk, 0)
        pltpu.sync_copy(lbuf_v, lu_hbm.at[b, h])
        for cch in range(4):
            idx_sl = idxg_v.at[pl.ds(cch * 512, 512)]
            pltpu.async_copy(so_hbm.at[idx_sl], rows_v, sem).wait()
            pltpu.sync_copy(rows_v, ou_hbm.at[b, h, pl.ds(cch * 512, 512)])
        return _

    lax.fori_loop(0, TPW, task_body, 0)


def _unsort_call(pos, so_flat, slog):
    mesh = plsc.VectorSubcoreMesh(core_axis_name="c", subcore_axis_name="s")
    fn = functools.partial(
        pl.kernel,
        mesh=mesh,
        compiler_params=pltpu.CompilerParams(
            needs_layout_passes=False, use_tc_tiling_on_sc=False),
        out_type=[
            jax.ShapeDtypeStruct((B, H, T, D), jnp.float32),
            jax.ShapeDtypeStruct((B, H, T), jnp.float32),
        ],
        scratch_types=[
            pltpu.VMEM((T,), jnp.int32),
            pltpu.VMEM((NT,), jnp.float32),
            pltpu.VMEM((T,), jnp.float32),
            pltpu.VMEM((T,), jnp.int32),
            pltpu.VMEM((512, D), jnp.float32),
            pltpu.SemaphoreType.DMA,
        ],
    )(_unsort_kernel)
    return fn(pos, so_flat, slog)


def _combine_body(l_ref, o_ref, out_ref):
    l = l_ref[0]
    mx = jnp.max(l, axis=0, keepdims=True)
    w = jnp.exp(l - mx)
    w = w / jnp.sum(w, axis=0, keepdims=True)
    acc = o_ref[0, 0] * w[0].reshape(T, 1)
    for h in range(1, H):
        acc = acc + o_ref[0, h] * w[h].reshape(T, 1)
    out_ref[0] = acc


def _combine_call(l_uns, o_uns):
    return pl.pallas_call(
        _combine_body,
        grid=(B,),
        in_specs=[
            pl.BlockSpec((1, H, T), lambda b: (b, 0, 0)),
            pl.BlockSpec((1, H, T, D), lambda b: (b, 0, 0, 0)),
        ],
        out_specs=pl.BlockSpec((1, T, D), lambda b: (b, 0, 0)),
        out_shape=jax.ShapeDtypeStruct((B, T, D), jnp.float32),
    )(l_uns, o_uns)


def kernel(qk, v, rotations):
    rot2 = rotations[0].reshape(D, H * 16)
    buck = _hash_call(qk, rot2)
    st, pos, sqk, sv = _sort_gather_call(
        buck, qk.reshape(B * T, D), v.reshape(B * T, D))
    stf = st.astype(jnp.float32)
    tq = stf.reshape(B, NT, 1)
    tk = stf.reshape(B, C, 1, BS)
    so, slog = _attn_call(sqk.reshape(B, C, BS, D), sv.reshape(B, C, BS, D), tq, tk)
    o_uns, l_uns = _unsort_call(pos, so.reshape(B * NT, D), slog.reshape(B, NT))
    return _combine_call(l_uns, o_uns)


# speedup vs baseline: 10.9937x; 1.4336x over previous
import functools

import jax
import jax.numpy as jnp
from jax import lax
from jax.experimental import pallas as pl
from jax.experimental.pallas import tpu as pltpu
from jax.experimental.pallas import tpu_sc as plsc

B, T, D = 16, 2048, 64
H = 8
NBUCK = 32
BS = 64
C = H * NBUCK
NT = H * T
NW = 32
TPW = (B * H) // NW
SELF_VAL = -50000.0
GCH = 256
DP = 2 * D


def _hash_body(qk_ref, v_ref, rot_ref, buck_ref, qkv_ref):
    x = qk_ref[0]
    rT = lax.dot_general(rot_ref[...], x, (((0,), (1,)), ((), ())),
                         preferred_element_type=jnp.float32)
    iota32 = lax.broadcasted_iota(jnp.int32, (NBUCK, T), 0)
    hrows = []
    for h in range(H):
        sub = rT[h * 16:(h + 1) * 16]
        seg = jnp.concatenate([sub, -sub], axis=0)
        m = jnp.max(seg, axis=0, keepdims=True)
        am = jnp.min(jnp.where(seg == m, iota32, NBUCK), axis=0, keepdims=True)
        blocks = [am[:, k * 128:(k + 1) * 128] for k in range(T // 128)]
        hrows.append(jnp.concatenate(blocks, axis=0).reshape(1, T // 128, 128))
    buck_ref[0] = jnp.concatenate(hrows, axis=0)
    qkv_ref[0] = jnp.concatenate([x, v_ref[0]], axis=1)


def _hash_call(qk, v, rot2):
    return pl.pallas_call(
        _hash_body,
        grid=(B,),
        in_specs=[
            pl.BlockSpec((1, T, D), lambda b: (b, 0, 0)),
            pl.BlockSpec((1, T, D), lambda b: (b, 0, 0)),
            pl.BlockSpec((D, H * 16), lambda b: (0, 0)),
        ],
        out_specs=[
            pl.BlockSpec((1, H, T // 128, 128), lambda b: (b, 0, 0, 0)),
            pl.BlockSpec((1, T, DP), lambda b: (b, 0, 0)),
        ],
        out_shape=[
            jax.ShapeDtypeStruct((B, H, T // 128, 128), jnp.int32),
            jax.ShapeDtypeStruct((B, T, DP), jnp.float32),
        ],
    )(qk, v, rot2)


def _sort_gather_kernel(buck_hbm, qkv_hbm,
                        st_hbm, pos_hbm, sqkv_hbm,
                        buk_v, rank_v, st_v, idxg_v, pos_v,
                        cnt_v, bs_v, rows_a, rows_b, sem_a, sem_b):
    wid = lax.axis_index("c") * 16 + lax.axis_index("s")
    idx16 = lax.iota(jnp.int32, 16)
    zeros16 = jnp.zeros((16,), jnp.int32)

    def task_body(i, _):
        tid = wid * TPW + i
        b = tid // H
        h = tid % H
        pltpu.sync_copy(buck_hbm.at[b, h], buk_v)
        cnt_v[pl.ds(0, 16)] = zeros16
        cnt_v[pl.ds(16, 16)] = zeros16

        def blk1(blk, _c):
            bvec = buk_v[pl.ds(blk * 16, 16)]
            key = bvec * 16 + idx16
            ks, vs = plsc.sort_key_val(key, idx16)
            bs = lax.shift_right_logical(ks, 4)
            bs_v[...] = bs
            prev = plsc.load_gather(bs_v, [jnp.maximum(idx16 - 1, 0)])
            is_start = (idx16 == 0) | (bs != prev)
            start_idx = plsc.cummax(jnp.where(is_start, idx16, 0))
            rnk = (idx16 - start_idx) + plsc.load_gather(cnt_v, [bs])
            nxt = plsc.load_gather(bs_v, [jnp.minimum(idx16 + 1, 15)])
            is_end = (idx16 == 15) | (bs != nxt)
            plsc.store_scatter(cnt_v, [bs], rnk + 1, mask=is_end)
            plsc.store_scatter(rank_v, [blk * 16 + vs], rnk)
            return _c

        lax.fori_loop(0, T // 16, blk1, 0)

        c0 = cnt_v[pl.ds(0, 16)]
        c1 = cnt_v[pl.ds(16, 16)]
        t0 = jnp.sum(c0)
        base0 = plsc.cumsum(c0) - c0
        base1 = plsc.cumsum(c1) + t0 - c1
        cnt_v[pl.ds(0, 16)] = base0
        cnt_v[pl.ds(16, 16)] = base1

        def blk2(blk, _c):
            bvec = buk_v[pl.ds(blk * 16, 16)]
            rnk = rank_v[pl.ds(blk * 16, 16)]
            ploc = plsc.load_gather(cnt_v, [bvec]) + rnk
            tvec = blk * 16 + idx16
            pos_v[pl.ds(blk * 16, 16)] = ploc + h * T
            plsc.store_scatter(st_v, [ploc], tvec)
            plsc.store_scatter(idxg_v, [ploc], tvec + b * T)
            return _c

        lax.fori_loop(0, T // 16, blk2, 0)

        pltpu.sync_copy(pos_v, pos_hbm.at[b, h])
        pltpu.sync_copy(st_v, st_hbm.at[b, pl.ds(h * T, T)])

        NCH = T // GCH
        bufs = (rows_a, rows_b)
        sems = (sem_a, sem_b)

        def fire(cch):
            idx_sl = idxg_v.at[pl.ds(cch * GCH, GCH)]
            return pltpu.async_copy(qkv_hbm.at[idx_sl], bufs[cch % 2], sems[cch % 2])

        cps = [fire(0), fire(1)]
        for cch in range(NCH):
            cps[cch].wait()
            pltpu.sync_copy(bufs[cch % 2],
                            sqkv_hbm.at[b, pl.ds(h * T + cch * GCH, GCH)])
            if cch + 2 < NCH:
                cps.append(fire(cch + 2))
        return _

    lax.fori_loop(0, TPW, task_body, 0)


def _sort_gather_call(buck, qkv_flat):
    mesh = plsc.VectorSubcoreMesh(core_axis_name="c", subcore_axis_name="s")
    fn = functools.partial(
        pl.kernel,
        mesh=mesh,
        compiler_params=pltpu.CompilerParams(
            needs_layout_passes=False, use_tc_tiling_on_sc=False),
        out_type=[
            jax.ShapeDtypeStruct((B, NT), jnp.int32),
            jax.ShapeDtypeStruct((B, H, T), jnp.int32),
            jax.ShapeDtypeStruct((B, NT, DP), jnp.float32),
        ],
        scratch_types=[
            pltpu.VMEM((T,), jnp.int32),
            pltpu.VMEM((T,), jnp.int32),
            pltpu.VMEM((T,), jnp.int32),
            pltpu.VMEM((T,), jnp.int32),
            pltpu.VMEM((T,), jnp.int32),
            pltpu.VMEM((32,), jnp.int32),
            pltpu.VMEM((16,), jnp.int32),
            pltpu.VMEM((GCH, DP), jnp.float32),
            pltpu.VMEM((GCH, DP), jnp.float32),
            pltpu.SemaphoreType.DMA,
            pltpu.SemaphoreType.DMA,
        ],
    )(_sort_gather_kernel)
    return fn(buck, qkv_flat)


def _attn_body(qc, qp1, tq, tkc, tkp, so_ref):
    mv = -jnp.finfo(jnp.float32).max
    NCH = 8
    R = NCH * BS
    qs = [qc[0, j, :, :D] for j in range(NCH)]
    vs = [qc[0, j, :, D:] for j in range(NCH)]
    qprev = qp1[0, 0, :, :D]
    vprev = qp1[0, 0, :, D:]
    dots_list = []
    for j in range(NCH):
        kmat = jnp.concatenate([qs[j], qprev if j == 0 else qs[j - 1]], axis=0)
        dots_list.append(
            lax.dot_general(qs[j], kmat, (((1,), (1,)), ((), ())),
                            preferred_element_type=jnp.float32))
    dots = jnp.concatenate(dots_list, axis=0)

    ns = jnp.sum(qc[0, :, :, :D] * qc[0, :, :, :D], axis=2)
    nprev = jnp.sum(qprev * qprev, axis=1).reshape(1, BS)
    nshift = jnp.concatenate([nprev, ns[:-1]], axis=0)
    nmat = jnp.concatenate([ns, nshift], axis=1)
    scale = 0.125 / jnp.maximum(jnp.sqrt(nmat), 1e-12)
    scale_b = jnp.broadcast_to(scale[:, None, :], (NCH, BS, 2 * BS)).reshape(R, 2 * BS)
    dots = dots * scale_b

    tcur = tkc[0, :, 0]
    tshift = jnp.concatenate([tkp[0, 0], tcur[:-1]], axis=0)
    ktm = jnp.concatenate([tcur, tshift], axis=1)
    ktb = jnp.broadcast_to(ktm[:, None, :], (NCH, BS, 2 * BS)).reshape(R, 2 * BS)
    qt = tq[0]
    dots = jnp.where(qt < ktb, mv, dots)
    dots = jnp.where(qt == ktb, SELF_VAL, dots)

    mx = jnp.max(dots, axis=1, keepdims=True)
    ex = jnp.exp(dots - mx)
    s = jnp.sum(ex, axis=1, keepdims=True)
    lse = jnp.log(s) + mx

    o_list = []
    for j in range(NCH):
        vmat = jnp.concatenate([vs[j], vprev if j == 0 else vs[j - 1]], axis=0)
        o_list.append(
            lax.dot_general(ex[j * BS:(j + 1) * BS], vmat,
                            (((1,), (0,)), ((), ())),
                            preferred_element_type=jnp.float32))
    ocat = jnp.concatenate(o_list, axis=0) / s
    so_ref[0] = jnp.concatenate(
        [ocat, jnp.broadcast_to(lse, (R, D))], axis=1)


def _attn_call(sqkv4, tq, tk):
    NI = C // 8
    R = 8 * BS
    cur4 = pl.BlockSpec((1, 8, BS, DP), lambda b, i: (b, i, 0, 0))
    prev1 = pl.BlockSpec((1, 1, BS, DP), lambda b, i: (b, (i * 8 + C - 1) % C, 0, 0))
    tqs = pl.BlockSpec((1, R, 1), lambda b, i: (b, i, 0))
    tkc = pl.BlockSpec((1, 8, 1, BS), lambda b, i: (b, i, 0, 0))
    tkp = pl.BlockSpec((1, 1, 1, BS), lambda b, i: (b, (i * 8 + C - 1) % C, 0, 0))
    return pl.pallas_call(
        _attn_body,
        grid=(B, NI),
        in_specs=[cur4, prev1, tqs, tkc, tkp],
        out_specs=pl.BlockSpec((1, R, DP), lambda b, i: (b, i, 0)),
        out_shape=jax.ShapeDtypeStruct((B, NT, DP), jnp.float32),
    )(sqkv4, sqkv4, tq, tk, tk)


def _unsort_kernel(pos_hbm, so_hbm,
                   ou_hbm,
                   pos_v, idxg_v, rows_a, rows_b, sem_a, sem_b):
    wid = lax.axis_index("c") * 16 + lax.axis_index("s")
    idx16 = lax.iota(jnp.int32, 16)

    def task_body(i, _):
        tid = wid * TPW + i
        b = tid // H
        h = tid % H
        pltpu.sync_copy(pos_hbm.at[b, h], pos_v)

        def blk(blk_i, _c):
            p = pos_v[pl.ds(blk_i * 16, 16)]
            idxg_v[pl.ds(blk_i * 16, 16)] = p + b * NT
            return _c

        lax.fori_loop(0, T // 16, blk, 0)
        NCH = T // GCH
        bufs = (rows_a, rows_b)
        sems = (sem_a, sem_b)

        def fire(cch):
            idx_sl = idxg_v.at[pl.ds(cch * GCH, GCH)]
            return pltpu.async_copy(so_hbm.at[idx_sl], bufs[cch % 2], sems[cch % 2])

        cps = [fire(0), fire(1)]
        for cch in range(NCH):
            cps[cch].wait()
            pltpu.sync_copy(bufs[cch % 2],
                            ou_hbm.at[b, h, pl.ds(cch * GCH, GCH)])
            if cch + 2 < NCH:
                cps.append(fire(cch + 2))
        return _

    lax.fori_loop(0, TPW, task_body, 0)


def _unsort_call(pos, so_flat):
    mesh = plsc.VectorSubcoreMesh(core_axis_name="c", subcore_axis_name="s")
    fn = functools.partial(
        pl.kernel,
        mesh=mesh,
        compiler_params=pltpu.CompilerParams(
            needs_layout_passes=False, use_tc_tiling_on_sc=False),
        out_type=jax.ShapeDtypeStruct((B, H, T, DP), jnp.float32),
        scratch_types=[
            pltpu.VMEM((T,), jnp.int32),
            pltpu.VMEM((T,), jnp.int32),
            pltpu.VMEM((GCH, DP), jnp.float32),
            pltpu.VMEM((GCH, DP), jnp.float32),
            pltpu.SemaphoreType.DMA,
            pltpu.SemaphoreType.DMA,
        ],
    )(_unsort_kernel)
    return fn(pos, so_flat)


def _combine_body(o_ref, out_ref):
    l = o_ref[0, :, :, D:D + 1]
    mx = jnp.max(l, axis=0, keepdims=True)
    w = jnp.exp(l - mx)
    w = w / jnp.sum(w, axis=0, keepdims=True)
    acc = o_ref[0, 0, :, :D] * w[0]
    for h in range(1, H):
        acc = acc + o_ref[0, h, :, :D] * w[h]
    out_ref[0] = acc


def _combine_call(o_uns):
    return pl.pallas_call(
        _combine_body,
        grid=(B,),
        in_specs=[pl.BlockSpec((1, H, T, DP), lambda b: (b, 0, 0, 0))],
        out_specs=pl.BlockSpec((1, T, D), lambda b: (b, 0, 0)),
        out_shape=jax.ShapeDtypeStruct((B, T, D), jnp.float32),
    )(o_uns)


def kernel(qk, v, rotations):
    rot2 = rotations[0].reshape(D, H * 16)
    buck4, qkv = _hash_call(qk, v, rot2)
    st, pos, sqkv = _sort_gather_call(
        buck4.reshape(B, H, T), qkv.reshape(B * T, DP))
    stf = st.astype(jnp.float32)
    tq = stf.reshape(B, NT, 1)
    tk = stf.reshape(B, C, 1, BS)
    so = _attn_call(sqkv.reshape(B, C, BS, DP), tq, tk)
    o_uns = _unsort_call(pos, so.reshape(B * NT, DP))
    return _combine_call(o_uns)


# speedup vs baseline: 12.0250x; 1.0938x over previous
import functools

import jax
import jax.numpy as jnp
from jax import lax
from jax.experimental import pallas as pl
from jax.experimental.pallas import tpu as pltpu
from jax.experimental.pallas import tpu_sc as plsc

B, T, D = 16, 2048, 64
H = 8
NBUCK = 32
BS = 64
C = H * NBUCK
NT = H * T
NW = 32
TPW = (B * H) // NW
SELF_VAL = -50000.0
GCH = 256
DP = 2 * D


def _hash_body(qk_ref, v_ref, rot_ref, buck_ref, qkv_ref, nrm_ref):
    x = qk_ref[0]
    rT = lax.dot_general(rot_ref[...], x, (((0,), (1,)), ((), ())),
                         preferred_element_type=jnp.float32)
    iota32 = lax.broadcasted_iota(jnp.int32, (NBUCK, T), 0)
    hrows = []
    for h in range(H):
        sub = rT[h * 16:(h + 1) * 16]
        seg = jnp.concatenate([sub, -sub], axis=0)
        m = jnp.max(seg, axis=0, keepdims=True)
        am = jnp.min(jnp.where(seg == m, iota32, NBUCK), axis=0, keepdims=True)
        blocks = [am[:, k * 128:(k + 1) * 128] for k in range(T // 128)]
        hrows.append(jnp.concatenate(blocks, axis=0).reshape(1, T // 128, 128))
    buck_ref[0] = jnp.concatenate(hrows, axis=0)
    xsq = x * x
    n_row = jnp.sum(xsq, axis=1, keepdims=True)
    qkn = x * (1.0 / jnp.maximum(jnp.sqrt(n_row), 1e-12))
    qkv_ref[0] = jnp.concatenate([qkn, v_ref[0]], axis=1)
    n_lane = jnp.sqrt(lax.dot_general(
        jnp.ones((1, D), jnp.float32), xsq, (((1,), (1,)), ((), ())),
        preferred_element_type=jnp.float32))
    nbl = [n_lane[:, k * 128:(k + 1) * 128] for k in range(T // 128)]
    nrm_ref[0] = jnp.concatenate(nbl, axis=0)


def _hash_call(qk, v, rot2):
    return pl.pallas_call(
        _hash_body,
        grid=(B,),
        in_specs=[
            pl.BlockSpec((1, T, D), lambda b: (b, 0, 0)),
            pl.BlockSpec((1, T, D), lambda b: (b, 0, 0)),
            pl.BlockSpec((D, H * 16), lambda b: (0, 0)),
        ],
        out_specs=[
            pl.BlockSpec((1, H, T // 128, 128), lambda b: (b, 0, 0, 0)),
            pl.BlockSpec((1, T, DP), lambda b: (b, 0, 0)),
            pl.BlockSpec((1, T // 128, 128), lambda b: (b, 0, 0)),
        ],
        out_shape=[
            jax.ShapeDtypeStruct((B, H, T // 128, 128), jnp.int32),
            jax.ShapeDtypeStruct((B, T, DP), jnp.float32),
            jax.ShapeDtypeStruct((B, T // 128, 128), jnp.float32),
        ],
    )(qk, v, rot2)


def _sort_gather_kernel(buck_hbm, qkv_hbm, nrm_hbm,
                        st_hbm, pos_hbm, sqkv_hbm, nst_hbm,
                        buk_v, rank_v, st_v, idxg_v, pos_v,
                        cnt_v, bs_v, nrm_v, nst_v, rows_a, rows_b, sem_a, sem_b):
    wid = lax.axis_index("c") * 16 + lax.axis_index("s")
    idx16 = lax.iota(jnp.int32, 16)
    zeros16 = jnp.zeros((16,), jnp.int32)

    def task_body(i, _):
        tid = wid * TPW + i
        b = tid // H
        h = tid % H
        pltpu.sync_copy(buck_hbm.at[b, h], buk_v)
        pltpu.sync_copy(nrm_hbm.at[b], nrm_v)
        cnt_v[pl.ds(0, 16)] = zeros16
        cnt_v[pl.ds(16, 16)] = zeros16

        def blk1(blk, _c):
            bvec = buk_v[pl.ds(blk * 16, 16)]
            key = bvec * 16 + idx16
            ks, vs = plsc.sort_key_val(key, idx16)
            bs = lax.shift_right_logical(ks, 4)
            bs_v[...] = bs
            prev = plsc.load_gather(bs_v, [jnp.maximum(idx16 - 1, 0)])
            is_start = (idx16 == 0) | (bs != prev)
            start_idx = plsc.cummax(jnp.where(is_start, idx16, 0))
            rnk = (idx16 - start_idx) + plsc.load_gather(cnt_v, [bs])
            nxt = plsc.load_gather(bs_v, [jnp.minimum(idx16 + 1, 15)])
            is_end = (idx16 == 15) | (bs != nxt)
            plsc.store_scatter(cnt_v, [bs], rnk + 1, mask=is_end)
            plsc.store_scatter(rank_v, [blk * 16 + vs], rnk)
            return _c

        lax.fori_loop(0, T // 16, blk1, 0)

        c0 = cnt_v[pl.ds(0, 16)]
        c1 = cnt_v[pl.ds(16, 16)]
        t0 = jnp.sum(c0)
        base0 = plsc.cumsum(c0) - c0
        base1 = plsc.cumsum(c1) + t0 - c1
        cnt_v[pl.ds(0, 16)] = base0
        cnt_v[pl.ds(16, 16)] = base1

        def blk2(blk, _c):
            bvec = buk_v[pl.ds(blk * 16, 16)]
            rnk = rank_v[pl.ds(blk * 16, 16)]
            ploc = plsc.load_gather(cnt_v, [bvec]) + rnk
            tvec = blk * 16 + idx16
            pos_v[pl.ds(blk * 16, 16)] = ploc + h * T
            plsc.store_scatter(st_v, [ploc], tvec)
            plsc.store_scatter(idxg_v, [ploc], tvec + b * T)
            plsc.store_scatter(nst_v, [ploc], nrm_v[pl.ds(blk * 16, 16)])
            return _c

        lax.fori_loop(0, T // 16, blk2, 0)

        pltpu.sync_copy(pos_v, pos_hbm.at[b, h])
        pltpu.sync_copy(st_v, st_hbm.at[b, pl.ds(h * T, T)])
        pltpu.sync_copy(nst_v, nst_hbm.at[b, pl.ds(h * T, T)])

        NCH = T // GCH
        bufs = (rows_a, rows_b)
        sems = (sem_a, sem_b)

        def fire(cch):
            idx_sl = idxg_v.at[pl.ds(cch * GCH, GCH)]
            return pltpu.async_copy(qkv_hbm.at[idx_sl], bufs[cch % 2], sems[cch % 2])

        cps = [fire(0), fire(1)]
        for cch in range(NCH):
            cps[cch].wait()
            pltpu.sync_copy(bufs[cch % 2],
                            sqkv_hbm.at[b, pl.ds(h * T + cch * GCH, GCH)])
            if cch + 2 < NCH:
                cps.append(fire(cch + 2))
        return _

    lax.fori_loop(0, TPW, task_body, 0)


def _sort_gather_call(buck, qkv_flat, nrm_flat):
    mesh = plsc.VectorSubcoreMesh(core_axis_name="c", subcore_axis_name="s")
    fn = functools.partial(
        pl.kernel,
        mesh=mesh,
        compiler_params=pltpu.CompilerParams(
            needs_layout_passes=False, use_tc_tiling_on_sc=False),
        out_type=[
            jax.ShapeDtypeStruct((B, NT), jnp.int32),
            jax.ShapeDtypeStruct((B, H, T), jnp.int32),
            jax.ShapeDtypeStruct((B, NT, DP), jnp.float32),
            jax.ShapeDtypeStruct((B, NT), jnp.float32),
        ],
        scratch_types=[
            pltpu.VMEM((T,), jnp.int32),
            pltpu.VMEM((T,), jnp.int32),
            pltpu.VMEM((T,), jnp.int32),
            pltpu.VMEM((T,), jnp.int32),
            pltpu.VMEM((T,), jnp.int32),
            pltpu.VMEM((32,), jnp.int32),
            pltpu.VMEM((16,), jnp.int32),
            pltpu.VMEM((T,), jnp.float32),
            pltpu.VMEM((T,), jnp.float32),
            pltpu.VMEM((GCH, DP), jnp.float32),
            pltpu.VMEM((GCH, DP), jnp.float32),
            pltpu.SemaphoreType.DMA,
            pltpu.SemaphoreType.DMA,
        ],
    )(_sort_gather_kernel)
    return fn(buck, qkv_flat, nrm_flat)


ACH = 16
AR = ACH * BS


def _attn_body(qc, qp1, tq, nq, tkc, tkp, so_ref):
    mv = -jnp.finfo(jnp.float32).max
    qs = [qc[0, j * BS:(j + 1) * BS, :D] for j in range(ACH)]
    vs = [qc[0, j * BS:(j + 1) * BS, D:] for j in range(ACH)]
    qprev = qp1[0, :, :D]
    vprev = qp1[0, :, D:]
    dots_list = []
    for j in range(ACH):
        kmat = jnp.concatenate([qs[j], qprev if j == 0 else qs[j - 1]], axis=0)
        dots_list.append(
            lax.dot_general(qs[j], kmat, (((1,), (1,)), ((), ())),
                            preferred_element_type=jnp.float32))
    dots = jnp.concatenate(dots_list, axis=0)
    dots = dots * (nq[0] * 0.125)

    tcur = tkc[0, :, 0]
    tshift = jnp.concatenate([tkp[0, 0], tcur[:-1]], axis=0)
    ktm = jnp.concatenate([tcur, tshift], axis=1)
    ktb = jnp.broadcast_to(ktm[:, None, :], (ACH, BS, 2 * BS)).reshape(AR, 2 * BS)
    qt = tq[0]
    dots = jnp.where(qt < ktb, mv, dots)
    dots = jnp.where(qt == ktb, SELF_VAL, dots)

    mx = jnp.max(dots, axis=1, keepdims=True)
    ex = jnp.exp(dots - mx)
    s = jnp.sum(ex, axis=1, keepdims=True)
    lse = jnp.log(s) + mx

    o_list = []
    for j in range(ACH):
        vmat = jnp.concatenate([vs[j], vprev if j == 0 else vs[j - 1]], axis=0)
        o_list.append(
            lax.dot_general(ex[j * BS:(j + 1) * BS], vmat,
                            (((1,), (0,)), ((), ())),
                            preferred_element_type=jnp.float32))
    ocat = jnp.concatenate(o_list, axis=0) / s
    so_ref[0] = jnp.concatenate(
        [ocat, jnp.broadcast_to(lse, (AR, D))], axis=1)


def _attn_call(sqkv, tq, nq, tk):
    NI = NT // AR
    NB64 = NT // BS
    cur = pl.BlockSpec((1, AR, DP), lambda b, i: (b, i, 0))
    prev = pl.BlockSpec((1, BS, DP),
                        lambda b, i: (b, (i * ACH + NB64 - 1) % NB64, 0))
    tqs = pl.BlockSpec((1, AR, 1), lambda b, i: (b, i, 0))
    tkc = pl.BlockSpec((1, ACH, 1, BS), lambda b, i: (b, i, 0, 0))
    tkp = pl.BlockSpec((1, 1, 1, BS),
                       lambda b, i: (b, (i * ACH + C - 1) % C, 0, 0))
    return pl.pallas_call(
        _attn_body,
        grid=(B, NI),
        in_specs=[cur, prev, tqs, tqs, tkc, tkp],
        out_specs=pl.BlockSpec((1, AR, DP), lambda b, i: (b, i, 0)),
        out_shape=jax.ShapeDtypeStruct((B, NT, DP), jnp.float32),
    )(sqkv, sqkv, tq, nq, tk, tk)


def _unsort_kernel(pos_hbm, so_hbm,
                   ou_hbm,
                   pos_v, idxg_v, rows_a, rows_b, sem_a, sem_b):
    wid = lax.axis_index("c") * 16 + lax.axis_index("s")
    idx16 = lax.iota(jnp.int32, 16)

    def task_body(i, _):
        tid = wid * TPW + i
        b = tid // H
        h = tid % H
        pltpu.sync_copy(pos_hbm.at[b, h], pos_v)

        def blk(blk_i, _c):
            p = pos_v[pl.ds(blk_i * 16, 16)]
            idxg_v[pl.ds(blk_i * 16, 16)] = p + b * NT
            return _c

        lax.fori_loop(0, T // 16, blk, 0)
        NCH = T // GCH
        bufs = (rows_a, rows_b)
        sems = (sem_a, sem_b)

        def fire(cch):
            idx_sl = idxg_v.at[pl.ds(cch * GCH, GCH)]
            return pltpu.async_copy(so_hbm.at[idx_sl], bufs[cch % 2], sems[cch % 2])

        cps = [fire(0), fire(1)]
        for cch in range(NCH):
            cps[cch].wait()
            pltpu.sync_copy(bufs[cch % 2],
                            ou_hbm.at[b, h, pl.ds(cch * GCH, GCH)])
            if cch + 2 < NCH:
                cps.append(fire(cch + 2))
        return _

    lax.fori_loop(0, TPW, task_body, 0)


def _unsort_call(pos, so_flat):
    mesh = plsc.VectorSubcoreMesh(core_axis_name="c", subcore_axis_name="s")
    fn = functools.partial(
        pl.kernel,
        mesh=mesh,
        compiler_params=pltpu.CompilerParams(
            needs_layout_passes=False, use_tc_tiling_on_sc=False),
        out_type=jax.ShapeDtypeStruct((B, H, T, DP), jnp.float32),
        scratch_types=[
            pltpu.VMEM((T,), jnp.int32),
            pltpu.VMEM((T,), jnp.int32),
            pltpu.VMEM((GCH, DP), jnp.float32),
            pltpu.VMEM((GCH, DP), jnp.float32),
            pltpu.SemaphoreType.DMA,
            pltpu.SemaphoreType.DMA,
        ],
    )(_unsort_kernel)
    return fn(pos, so_flat)


def _combine_body(o_ref, out_ref):
    l = o_ref[0, :, :, D:D + 1]
    mx = jnp.max(l, axis=0, keepdims=True)
    w = jnp.exp(l - mx)
    w = w / jnp.sum(w, axis=0, keepdims=True)
    acc = o_ref[0, 0, :, :D] * w[0]
    for h in range(1, H):
        acc = acc + o_ref[0, h, :, :D] * w[h]
    out_ref[0] = acc


def _combine_call(o_uns):
    return pl.pallas_call(
        _combine_body,
        grid=(B,),
        in_specs=[pl.BlockSpec((1, H, T, DP), lambda b: (b, 0, 0, 0))],
        out_specs=pl.BlockSpec((1, T, D), lambda b: (b, 0, 0)),
        out_shape=jax.ShapeDtypeStruct((B, T, D), jnp.float32),
    )(o_uns)


def kernel(qk, v, rotations):
    rot2 = rotations[0].reshape(D, H * 16)
    buck4, qkv, nrm = _hash_call(qk, v, rot2)
    st, pos, sqkv, nst = _sort_gather_call(
        buck4.reshape(B, H, T), qkv.reshape(B * T, DP), nrm.reshape(B, T))
    stf = st.astype(jnp.float32)
    tq = stf.reshape(B, NT, 1)
    nq = nst.reshape(B, NT, 1)
    tk = stf.reshape(B, C, 1, BS)
    so = _attn_call(sqkv, tq, nq, tk)
    o_uns = _unsort_call(pos, so.reshape(B * NT, DP))
    return _combine_call(o_uns)
